# Initial kernel scaffold; baseline (speedup 1.0000x reference)
import jax
import jax.numpy as jnp
from jax.experimental import pallas as pl


def kernel(stacked_radii, ptypes, lattices, gt_OO, gt_HH, gt_HO):
    raise NotImplementedError("write your pallas kernel here")


# speedup vs baseline: 1557.9120x; 1557.9120x over previous
import functools

import numpy as np
import jax
import jax.numpy as jnp
from jax import lax
from jax.experimental import pallas as pl
from jax.experimental.pallas import tpu as pltpu
from jax.experimental.pallas import tpu_sc as plsc

_XLIM = 6.0
_NB = 600
_BINS = np.linspace(1e-06, _XLIM, _NB + 1).astype(np.float32)
_SHELL = (4.0 / 3.0) * np.pi * (_BINS[1:] ** 3 - _BINS[:-1] ** 3)
_H64 = (_XLIM - 1e-06) / _NB
_HS = np.float32(_H64)
_INV_H = np.float32(1.0 / _H64)
_A0 = np.float32(_BINS[0])
_BMAX = np.float32(_BINS[-1])

_NLANE = 16
_NCORE = 2
_NSUB = 16
_NWORKER = _NCORE * _NSUB


def _idx_body(nrep, n_o, dump, lat_ref, prow_ref, pcol_ref, out_ref):
    n = prow_ref.shape[1]
    acc = jnp.zeros((n, n), jnp.float32)
    for k in range(3):
        lk = lat_ref[k]
        a = prow_ref[0, :, pl.ds(k, 1)]
        b = pcol_ref[0, pl.ds(k, 1), :]
        aw = (a / lk % 1.0) * lk
        bw = (b / lk % 1.0) * lk
        delta = jnp.abs(aw - bw)
        delta = jnp.where(delta > 0.5 * lk, delta - lk, delta)
        acc = acc + delta * delta
    d = jnp.sqrt(acc)
    jf = jnp.floor((d - _A0) * _INV_H)
    e_lo = jf * _HS + _A0
    e_hi = (jf + 1.0) * _HS + _A0
    j = jf.astype(jnp.int32)
    j = j + (d >= e_hi).astype(jnp.int32) - (d < e_lo).astype(jnp.int32)
    j = jnp.clip(j, 0, _NB - 1)
    valid = (d >= _A0) & (d <= _BMAX)
    ri = lax.broadcasted_iota(jnp.int32, (n, n), 0)
    ci = lax.broadcasted_iota(jnp.int32, (n, n), 1)
    is_oa = ri < n_o
    is_ob = ci < n_o
    off = jnp.where(
        is_oa & is_ob, 0,
        jnp.where((~is_oa) & (~is_ob), _NB,
                  jnp.where(is_oa & (~is_ob), 2 * _NB, -1)))
    valid = valid & (off >= 0)
    rep = pl.program_id(0) % nrep
    combined = rep * (3 * _NB) + off + j
    out_ref[0] = jnp.where(valid, combined, dump).astype(jnp.int32)


def _make_idx_call(f, n, nrep, n_o, dump):
    return pl.pallas_call(
        functools.partial(_idx_body, nrep, n_o, dump),
        grid=(f,),
        in_specs=[
            pl.BlockSpec(memory_space=pltpu.SMEM),
            pl.BlockSpec((1, n, 3), lambda i: (i, 0, 0)),
            pl.BlockSpec((1, 3, n), lambda i: (i, 0, 0)),
        ],
        out_specs=pl.BlockSpec((1, n, n), lambda i: (i, 0, 0)),
        out_shape=jax.ShapeDtypeStruct((f, n, n), jnp.int32),
    )


def _sc_hist_body(perw, ch, accw, idx_hbm, out_hbm, acc_ref, buf_ref):
    wid = lax.axis_index("s") * _NCORE + lax.axis_index("c")
    lane_off = lax.iota(jnp.int32, 16) * accw
    ones = jnp.ones((16,), jnp.float32)
    zeros = jnp.zeros((16,), jnp.float32)

    def zbody(i, _):
        acc_ref[pl.ds(i * 16, 16)] = zeros
        return 0
    lax.fori_loop(0, (_NLANE * accw) // 16, zbody, 0)

    def chunk_body(ci, _):
        start = wid * perw + ci * ch
        pltpu.sync_copy(idx_hbm.at[pl.ds(start, ch)], buf_ref)

        def vbody(vi, _):
            v = buf_ref[pl.ds(vi * 16, 16)]
            plsc.addupdate_scatter(acc_ref, [lane_off + v], ones)
            return 0
        lax.fori_loop(0, ch // 16, vbody, 0)
        return 0
    lax.fori_loop(0, perw // ch, chunk_body, 0)

    def rbody(i, _):
        s = acc_ref[pl.ds(i * 16, 16)]
        for rr in range(1, _NLANE):
            s = s + acc_ref[pl.ds(rr * accw + i * 16, 16)]
        acc_ref[pl.ds(i * 16, 16)] = s
        return 0
    lax.fori_loop(0, accw // 16, rbody, 0)

    pltpu.sync_copy(acc_ref.at[pl.ds(0, accw)],
                    out_hbm.at[pl.ds(wid * accw, accw)])


def _make_sc_hist(tot, accw):
    perw = tot // _NWORKER
    ch = 7200
    while perw % ch != 0:
        ch //= 2
    mesh = plsc.VectorSubcoreMesh(
        core_axis_name="c", subcore_axis_name="s", num_cores=_NCORE)
    return pl.kernel(
        functools.partial(_sc_hist_body, perw, ch, accw),
        out_type=jax.ShapeDtypeStruct((_NWORKER * accw,), jnp.float32),
        mesh=mesh,
        compiler_params=pltpu.CompilerParams(needs_layout_passes=False),
        scratch_types=[
            pltpu.VMEM((_NLANE * accw,), jnp.float32),
            pltpu.VMEM((ch,), jnp.int32),
        ],
    )


def _norm_body(t, nrep, n_o, n_h, lat_ref, parts_ref, gts_ref, shell_ref,
               rdf_ref, maes_ref):
    tot = jnp.sum(parts_ref[...], axis=0, keepdims=True)
    prod_l = lat_ref[0] * lat_ref[1] * lat_ref[2]
    counts = (n_o * n_o, n_h * n_h, n_o * n_h)
    shell = shell_ref[...]
    li = lax.broadcasted_iota(jnp.int32, (1, 128), 1)
    mvec = jnp.zeros((1, 128), jnp.float32)
    for rep in range(nrep):
        maes = []
        for c in range(3):
            h = lax.slice(tot, (0, rep * 3 * _NB + c * _NB),
                          (1, rep * 3 * _NB + (c + 1) * _NB))
            data_shape = jnp.float32(t) * jnp.float32(counts[c])
            rho = data_shape / prod_l
            z = rho * shell
            rdf = h / z
            rdf_ref[pl.ds(rep * 3 + c, 1), :] = rdf
            g = gts_ref[pl.ds(c, 1), :]
            maes.append(_XLIM * (jnp.sum(jnp.abs(rdf - g)) / jnp.float32(_NB)))
        m = jnp.maximum(jnp.maximum(maes[0], maes[1]), maes[2])
        mvec = jnp.where(li == rep, m, mvec)
    maes_ref[...] = mvec


def _make_norm_call(t, nrep, n_o, n_h, accw):
    return pl.pallas_call(
        functools.partial(_norm_body, t, nrep, n_o, n_h),
        in_specs=[
            pl.BlockSpec(memory_space=pltpu.SMEM),
            pl.BlockSpec((_NWORKER, accw), lambda: (0, 0)),
            pl.BlockSpec((3, _NB), lambda: (0, 0)),
            pl.BlockSpec((1, _NB), lambda: (0, 0)),
        ],
        out_specs=[
            pl.BlockSpec((3 * nrep, _NB), lambda: (0, 0)),
            pl.BlockSpec((1, 128), lambda: (0, 0)),
        ],
        out_shape=[
            jax.ShapeDtypeStruct((3 * nrep, _NB), jnp.float32),
            jax.ShapeDtypeStruct((1, 128), jnp.float32),
        ],
    )


def kernel(stacked_radii, ptypes, lattices, gt_OO, gt_HH, gt_HO):
    t, nrep, n, _ = stacked_radii.shape
    f = t * nrep
    n_o = n // 3
    n_h = n - n_o
    dump = nrep * 3 * _NB
    accw = dump + (16 - dump % 16) % 16 + 16

    pos = stacked_radii.reshape(f, n, 3)
    pos2 = jnp.concatenate([pos[:, 0::3], pos[:, 1::3], pos[:, 2::3]], axis=1)
    pcol = jnp.transpose(pos2, (0, 2, 1))

    idx = _make_idx_call(f, n, nrep, n_o, dump)(lattices, pos2, pcol)
    parts = _make_sc_hist(f * n * n, accw)(idx.reshape(-1))
    parts = parts.reshape(_NWORKER, accw)

    gts = jnp.concatenate([gt_OO, gt_HH, gt_HO], axis=0)
    shell = jnp.asarray(_SHELL.astype(np.float32))[None, :]
    rdf12, maes_pad = _make_norm_call(t, nrep, n_o, n_h, accw)(
        lattices, parts, gts, shell)
    return rdf12.reshape(nrep, 3 * _NB), maes_pad[0, :nrep]


# speedup vs baseline: 1716.6343x; 1.1019x over previous
import functools

import numpy as np
import jax
import jax.numpy as jnp
from jax import lax
from jax.experimental import pallas as pl
from jax.experimental.pallas import tpu as pltpu
from jax.experimental.pallas import tpu_sc as plsc

_XLIM = 6.0
_NB = 600
_BINS = np.linspace(1e-06, _XLIM, _NB + 1).astype(np.float32)
_SHELL = (4.0 / 3.0) * np.pi * (_BINS[1:] ** 3 - _BINS[:-1] ** 3)
_H64 = (_XLIM - 1e-06) / _NB
_HS = np.float32(_H64)
_INV_H = np.float32(1.0 / _H64)
_A0 = np.float32(_BINS[0])
_BMAX = np.float32(_BINS[-1])

_NLANE = 16
_NCORE = 2
_NSUB = 16
_NWORKER = _NCORE * _NSUB


def _idx_body(nrep, n_o, dump, lat_ref, prow_ref, pcol_ref, out_ref):
    n = prow_ref.shape[1]
    acc = jnp.zeros((n, n), jnp.float32)
    for k in range(3):
        lk = lat_ref[k]
        a = prow_ref[0, :, pl.ds(k, 1)]
        b = pcol_ref[0, pl.ds(k, 1), :]
        aw = (a / lk % 1.0) * lk
        bw = (b / lk % 1.0) * lk
        delta = jnp.abs(aw - bw)
        delta = jnp.where(delta > 0.5 * lk, delta - lk, delta)
        acc = acc + delta * delta
    d = jnp.sqrt(acc)
    jf = jnp.floor((d - _A0) * _INV_H)
    e_lo = jf * _HS + _A0
    e_hi = (jf + 1.0) * _HS + _A0
    j = jf.astype(jnp.int32)
    j = j + (d >= e_hi).astype(jnp.int32) - (d < e_lo).astype(jnp.int32)
    j = jnp.clip(j, 0, _NB - 1)
    valid = (d >= _A0) & (d <= _BMAX)
    ri = lax.broadcasted_iota(jnp.int32, (n, n), 0)
    ci = lax.broadcasted_iota(jnp.int32, (n, n), 1)
    is_oa = ri < n_o
    is_ob = ci < n_o
    off = jnp.where(
        is_oa & is_ob, 0,
        jnp.where((~is_oa) & (~is_ob), _NB,
                  jnp.where(is_oa & (~is_ob), 2 * _NB, -1)))
    valid = valid & (off >= 0)
    rep = pl.program_id(0) % nrep
    combined = rep * (3 * _NB) + off + j
    out_ref[0] = jnp.where(valid, combined, dump).astype(jnp.int32)


def _make_idx_call(f, n, nrep, n_o, dump):
    return pl.pallas_call(
        functools.partial(_idx_body, nrep, n_o, dump),
        grid=(f,),
        in_specs=[
            pl.BlockSpec(memory_space=pltpu.SMEM),
            pl.BlockSpec((1, n, 3), lambda i: (i, 0, 0)),
            pl.BlockSpec((1, 3, n), lambda i: (i, 0, 0)),
        ],
        out_specs=pl.BlockSpec((1, n, n), lambda i: (i, 0, 0)),
        out_shape=jax.ShapeDtypeStruct((f, n, n), jnp.int32),
    )


def _sc_hist_body(perw, ch, accw, idx_hbm, out_hbm, acc_ref, buf_ref,
                  sem0, sem1):
    wid = lax.axis_index("s") * _NCORE + lax.axis_index("c")
    lane_off = lax.iota(jnp.int32, 16) * accw
    ones = jnp.ones((16,), jnp.float32)
    zeros = jnp.zeros((16,), jnp.float32)

    def zbody(i, _):
        acc_ref[pl.ds(i * 16, 16)] = zeros
        return 0
    lax.fori_loop(0, (_NLANE * accw) // 16, zbody, 0, unroll=8)

    nchunk = perw // ch
    base = wid * perw

    def copy_in(ci, slot, sem):
        return pltpu.async_copy(
            idx_hbm.at[pl.ds(base + ci * ch, ch)],
            buf_ref.at[pl.ds(slot * ch, ch)], sem)

    copy_in(0, 0, sem0)

    def chunk_body(oi, _):
        for b in range(2):
            ci = oi * 2 + b
            sem = sem0 if b == 0 else sem1
            nsem = sem1 if b == 0 else sem0

            @pl.when(ci + 1 < nchunk)
            def _():
                copy_in(ci + 1, 1 - b, nsem)

            pltpu.make_async_copy(
                idx_hbm.at[pl.ds(base, ch)],
                buf_ref.at[pl.ds(b * ch, ch)], sem).wait()

            def vbody(vi, _):
                v = buf_ref[pl.ds(b * ch + vi * 16, 16)]
                plsc.addupdate_scatter(acc_ref, [lane_off + v], ones)
                return 0
            lax.fori_loop(0, ch // 16, vbody, 0, unroll=10)
        return 0
    lax.fori_loop(0, nchunk // 2, chunk_body, 0)

    def rbody(i, _):
        s = acc_ref[pl.ds(i * 16, 16)]
        for rr in range(1, _NLANE):
            s = s + acc_ref[pl.ds(rr * accw + i * 16, 16)]
        acc_ref[pl.ds(i * 16, 16)] = s
        return 0
    lax.fori_loop(0, accw // 16, rbody, 0, unroll=2)

    pltpu.sync_copy(acc_ref.at[pl.ds(0, accw)],
                    out_hbm.at[pl.ds(wid * accw, accw)])


def _make_sc_hist(tot, accw):
    perw = tot // _NWORKER
    ch = 7200
    while perw % ch != 0 or (perw // ch) % 2 != 0:
        ch //= 2
    mesh = plsc.VectorSubcoreMesh(
        core_axis_name="c", subcore_axis_name="s", num_cores=_NCORE)
    return pl.kernel(
        functools.partial(_sc_hist_body, perw, ch, accw),
        out_type=jax.ShapeDtypeStruct((_NWORKER * accw,), jnp.float32),
        mesh=mesh,
        compiler_params=pltpu.CompilerParams(needs_layout_passes=False),
        scratch_types=[
            pltpu.VMEM((_NLANE * accw,), jnp.float32),
            pltpu.VMEM((2 * ch,), jnp.int32),
            pltpu.SemaphoreType.DMA,
            pltpu.SemaphoreType.DMA,
        ],
    )


def _norm_body(t, nrep, n_o, n_h, lat_ref, parts_ref, gts_ref, shell_ref,
               rdf_ref, maes_ref):
    tot = jnp.sum(parts_ref[...], axis=0, keepdims=True)
    prod_l = lat_ref[0] * lat_ref[1] * lat_ref[2]
    counts = (n_o * n_o, n_h * n_h, n_o * n_h)
    shell = shell_ref[...]
    li = lax.broadcasted_iota(jnp.int32, (1, 128), 1)
    mvec = jnp.zeros((1, 128), jnp.float32)
    for rep in range(nrep):
        maes = []
        for c in range(3):
            h = lax.slice(tot, (0, rep * 3 * _NB + c * _NB),
                          (1, rep * 3 * _NB + (c + 1) * _NB))
            data_shape = jnp.float32(t) * jnp.float32(counts[c])
            rho = data_shape / prod_l
            z = rho * shell
            rdf = h / z
            rdf_ref[pl.ds(rep * 3 + c, 1), :] = rdf
            g = gts_ref[pl.ds(c, 1), :]
            maes.append(_XLIM * (jnp.sum(jnp.abs(rdf - g)) / jnp.float32(_NB)))
        m = jnp.maximum(jnp.maximum(maes[0], maes[1]), maes[2])
        mvec = jnp.where(li == rep, m, mvec)
    maes_ref[...] = mvec


def _make_norm_call(t, nrep, n_o, n_h, accw):
    return pl.pallas_call(
        functools.partial(_norm_body, t, nrep, n_o, n_h),
        in_specs=[
            pl.BlockSpec(memory_space=pltpu.SMEM),
            pl.BlockSpec((_NWORKER, accw), lambda: (0, 0)),
            pl.BlockSpec((3, _NB), lambda: (0, 0)),
            pl.BlockSpec((1, _NB), lambda: (0, 0)),
        ],
        out_specs=[
            pl.BlockSpec((3 * nrep, _NB), lambda: (0, 0)),
            pl.BlockSpec((1, 128), lambda: (0, 0)),
        ],
        out_shape=[
            jax.ShapeDtypeStruct((3 * nrep, _NB), jnp.float32),
            jax.ShapeDtypeStruct((1, 128), jnp.float32),
        ],
    )


def kernel(stacked_radii, ptypes, lattices, gt_OO, gt_HH, gt_HO):
    t, nrep, n, _ = stacked_radii.shape
    f = t * nrep
    n_o = n // 3
    n_h = n - n_o
    dump = nrep * 3 * _NB
    accw = dump + (16 - dump % 16) % 16 + 16

    pos = stacked_radii.reshape(f, n, 3)
    pos2 = jnp.concatenate([pos[:, 0::3], pos[:, 1::3], pos[:, 2::3]], axis=1)
    pcol = jnp.transpose(pos2, (0, 2, 1))

    idx = _make_idx_call(f, n, nrep, n_o, dump)(lattices, pos2, pcol)
    parts = _make_sc_hist(f * n * n, accw)(idx.reshape(-1))
    parts = parts.reshape(_NWORKER, accw)

    gts = jnp.concatenate([gt_OO, gt_HH, gt_HO], axis=0)
    shell = jnp.asarray(_SHELL.astype(np.float32))[None, :]
    rdf12, maes_pad = _make_norm_call(t, nrep, n_o, n_h, accw)(
        lattices, parts, gts, shell)
    return rdf12.reshape(nrep, 3 * _NB), maes_pad[0, :nrep]


# speedup vs baseline: 2150.2017x; 1.2526x over previous
import functools

import numpy as np
import jax
import jax.numpy as jnp
from jax import lax
from jax.experimental import pallas as pl
from jax.experimental.pallas import tpu as pltpu
from jax.experimental.pallas import tpu_sc as plsc

_XLIM = 6.0
_NB = 600
_BINS = np.linspace(1e-06, _XLIM, _NB + 1).astype(np.float32)
_SHELL = (4.0 / 3.0) * np.pi * (_BINS[1:] ** 3 - _BINS[:-1] ** 3)
_H64 = (_XLIM - 1e-06) / _NB
_HS = np.float32(_H64)
_INV_H = np.float32(1.0 / _H64)
_A0 = np.float32(_BINS[0])
_BMAX = np.float32(_BINS[-1])

_NLANE = 16
_NCORE = 2
_NSUB = 16
_NWORKER = _NCORE * _NSUB


def _idx_body(nrep, n_o, dump, lat_ref, prow_ref, pcol_ref, out_ref):
    n = prow_ref.shape[1]
    acc = jnp.zeros((n, n), jnp.float32)
    for k in range(3):
        lk = lat_ref[k]
        a = prow_ref[0, :, pl.ds(k, 1)]
        b = pcol_ref[0, pl.ds(k, 1), :]
        aw = (a / lk % 1.0) * lk
        bw = (b / lk % 1.0) * lk
        delta = jnp.abs(aw - bw)
        delta = jnp.where(delta > 0.5 * lk, delta - lk, delta)
        acc = acc + delta * delta
    d = jnp.sqrt(acc)
    jf = jnp.floor((d - _A0) * _INV_H)
    e_lo = jf * _HS + _A0
    e_hi = (jf + 1.0) * _HS + _A0
    j = jf.astype(jnp.int32)
    j = j + (d >= e_hi).astype(jnp.int32) - (d < e_lo).astype(jnp.int32)
    j = jnp.clip(j, 0, _NB - 1)
    valid = (d >= _A0) & (d <= _BMAX)
    ri = lax.broadcasted_iota(jnp.int32, (n, n), 0)
    ci = lax.broadcasted_iota(jnp.int32, (n, n), 1)
    is_oa = ri < n_o
    is_ob = ci < n_o
    off = jnp.where(
        is_oa & is_ob, 0,
        jnp.where((~is_oa) & (~is_ob), _NB,
                  jnp.where(is_oa & (~is_ob), 2 * _NB, -1)))
    valid = valid & (off >= 0)
    rep = pl.program_id(0) % nrep
    combined = rep * (3 * _NB) + off + j
    out_ref[0] = jnp.where(valid, combined, dump).astype(jnp.int32)


def _make_idx_call(f, n, nrep, n_o, dump):
    return pl.pallas_call(
        functools.partial(_idx_body, nrep, n_o, dump),
        grid=(f,),
        in_specs=[
            pl.BlockSpec(memory_space=pltpu.SMEM),
            pl.BlockSpec((1, n, 3), lambda i: (i, 0, 0)),
            pl.BlockSpec((1, 3, n), lambda i: (i, 0, 0)),
        ],
        out_specs=pl.BlockSpec((1, n, n), lambda i: (i, 0, 0)),
        out_shape=jax.ShapeDtypeStruct((f, n, n), jnp.int32),
    )


def _sc_hist_body(perw, ch, accw, idx_hbm, out_hbm, acc_ref, buf_ref,
                  sem0, sem1):
    wid = lax.axis_index("s") * _NCORE + lax.axis_index("c")
    lane_off = lax.iota(jnp.int32, 16) * accw
    ones = jnp.ones((16,), jnp.float32)
    zeros = jnp.zeros((16,), jnp.float32)

    @plsc.parallel_loop(0, _NLANE * accw, 16, unroll=8)
    def _(i):
        acc_ref[pl.ds(i, 16)] = zeros

    nchunk = perw // ch
    base = wid * perw

    def copy_in(ci, slot, sem):
        return pltpu.async_copy(
            idx_hbm.at[pl.ds(base + ci * ch, ch)],
            buf_ref.at[pl.ds(slot * ch, ch)], sem)

    copy_in(0, 0, sem0)

    def chunk_body(oi, _):
        for b in range(2):
            ci = oi * 2 + b
            sem = sem0 if b == 0 else sem1
            nsem = sem1 if b == 0 else sem0

            @pl.when(ci + 1 < nchunk)
            def _():
                copy_in(ci + 1, 1 - b, nsem)

            pltpu.make_async_copy(
                idx_hbm.at[pl.ds(base, ch)],
                buf_ref.at[pl.ds(b * ch, ch)], sem).wait()

            @plsc.parallel_loop(0, ch, 16, unroll=10)
            def _(vi):
                v = buf_ref[pl.ds(b * ch + vi, 16)]
                plsc.addupdate_scatter(acc_ref, [lane_off + v], ones)
        return 0
    lax.fori_loop(0, nchunk // 2, chunk_body, 0)

    @plsc.parallel_loop(0, accw, 16, unroll=2)
    def _(i):
        s = acc_ref[pl.ds(i, 16)]
        for rr in range(1, _NLANE):
            s = s + acc_ref[pl.ds(rr * accw + i, 16)]
        acc_ref[pl.ds(i, 16)] = s

    pltpu.sync_copy(acc_ref.at[pl.ds(0, accw)],
                    out_hbm.at[pl.ds(wid * accw, accw)])


def _make_sc_hist(tot, accw):
    perw = tot // _NWORKER
    ch = 7200
    while perw % ch != 0 or (perw // ch) % 2 != 0:
        ch //= 2
    mesh = plsc.VectorSubcoreMesh(
        core_axis_name="c", subcore_axis_name="s", num_cores=_NCORE)
    return pl.kernel(
        functools.partial(_sc_hist_body, perw, ch, accw),
        out_type=jax.ShapeDtypeStruct((_NWORKER * accw,), jnp.float32),
        mesh=mesh,
        compiler_params=pltpu.CompilerParams(needs_layout_passes=False),
        scratch_types=[
            pltpu.VMEM((_NLANE * accw,), jnp.float32),
            pltpu.VMEM((2 * ch,), jnp.int32),
            pltpu.SemaphoreType.DMA,
            pltpu.SemaphoreType.DMA,
        ],
    )


def _norm_body(t, nrep, n_o, n_h, lat_ref, parts_ref, gts_ref, shell_ref,
               rdf_ref, maes_ref):
    tot = jnp.sum(parts_ref[...], axis=0, keepdims=True)
    prod_l = lat_ref[0] * lat_ref[1] * lat_ref[2]
    counts = (n_o * n_o, n_h * n_h, n_o * n_h)
    shell = shell_ref[...]
    li = lax.broadcasted_iota(jnp.int32, (1, 128), 1)
    mvec = jnp.zeros((1, 128), jnp.float32)
    for rep in range(nrep):
        maes = []
        for c in range(3):
            h = lax.slice(tot, (0, rep * 3 * _NB + c * _NB),
                          (1, rep * 3 * _NB + (c + 1) * _NB))
            data_shape = jnp.float32(t) * jnp.float32(counts[c])
            rho = data_shape / prod_l
            z = rho * shell
            rdf = h / z
            rdf_ref[pl.ds(rep * 3 + c, 1), :] = rdf
            g = gts_ref[pl.ds(c, 1), :]
            maes.append(_XLIM * (jnp.sum(jnp.abs(rdf - g)) / jnp.float32(_NB)))
        m = jnp.maximum(jnp.maximum(maes[0], maes[1]), maes[2])
        mvec = jnp.where(li == rep, m, mvec)
    maes_ref[...] = mvec


def _make_norm_call(t, nrep, n_o, n_h, accw):
    return pl.pallas_call(
        functools.partial(_norm_body, t, nrep, n_o, n_h),
        in_specs=[
            pl.BlockSpec(memory_space=pltpu.SMEM),
            pl.BlockSpec((_NWORKER, accw), lambda: (0, 0)),
            pl.BlockSpec((3, _NB), lambda: (0, 0)),
            pl.BlockSpec((1, _NB), lambda: (0, 0)),
        ],
        out_specs=[
            pl.BlockSpec((3 * nrep, _NB), lambda: (0, 0)),
            pl.BlockSpec((1, 128), lambda: (0, 0)),
        ],
        out_shape=[
            jax.ShapeDtypeStruct((3 * nrep, _NB), jnp.float32),
            jax.ShapeDtypeStruct((1, 128), jnp.float32),
        ],
    )


def kernel(stacked_radii, ptypes, lattices, gt_OO, gt_HH, gt_HO):
    t, nrep, n, _ = stacked_radii.shape
    f = t * nrep
    n_o = n // 3
    n_h = n - n_o
    dump = nrep * 3 * _NB
    accw = dump + (16 - dump % 16) % 16 + 16

    pos = stacked_radii.reshape(f, n, 3)
    pos2 = jnp.concatenate([pos[:, 0::3], pos[:, 1::3], pos[:, 2::3]], axis=1)
    pcol = jnp.transpose(pos2, (0, 2, 1))

    idx = _make_idx_call(f, n, nrep, n_o, dump)(lattices, pos2, pcol)
    parts = _make_sc_hist(f * n * n, accw)(idx.reshape(-1))
    parts = parts.reshape(_NWORKER, accw)

    gts = jnp.concatenate([gt_OO, gt_HH, gt_HO], axis=0)
    shell = jnp.asarray(_SHELL.astype(np.float32))[None, :]
    rdf12, maes_pad = _make_norm_call(t, nrep, n_o, n_h, accw)(
        lattices, parts, gts, shell)
    return rdf12.reshape(nrep, 3 * _NB), maes_pad[0, :nrep]


# speedup vs baseline: 2769.8176x; 1.2882x over previous
import functools

import numpy as np
import jax
import jax.numpy as jnp
from jax import lax
from jax.experimental import pallas as pl
from jax.experimental.pallas import tpu as pltpu
from jax.experimental.pallas import tpu_sc as plsc

_XLIM = 6.0
_NB = 600
_BINS = np.linspace(1e-06, _XLIM, _NB + 1).astype(np.float32)
_SHELL = (4.0 / 3.0) * np.pi * (_BINS[1:] ** 3 - _BINS[:-1] ** 3)
_H64 = (_XLIM - 1e-06) / _NB
_HS = np.float32(_H64)
_INV_H = np.float32(1.0 / _H64)
_A0 = np.float32(_BINS[0])
_BMAX = np.float32(_BINS[-1])

_NLANE = 16
_NCORE = 2
_NSUB = 16
_NWORKER = _NCORE * _NSUB


def _idx_body(fb, nrep, dump, lat_ref, prow_ref, pcol_ref, off_ref, out_ref):
    n = prow_ref.shape[1]
    offmap = off_ref[...]
    for b in range(fb):
        acc = jnp.zeros((n, n), jnp.float32)
        for k in range(3):
            lk = lat_ref[k]
            pa = prow_ref[b, :, pl.ds(k, 1)]
            pb = pcol_ref[b, pl.ds(k, 1), :]
            aw = (pa / lk % 1.0) * lk
            bw = (pb / lk % 1.0) * lk
            delta = jnp.abs(aw - bw)
            delta = jnp.where(delta > 0.5 * lk, delta - lk, delta)
            acc = acc + delta * delta
        d = jnp.sqrt(acc)
        j = jnp.floor((d - _A0) * _INV_H).astype(jnp.int32)
        j = jnp.clip(j, 0, _NB - 1)
        base = (b % nrep) * (3 * _NB)
        comb = jnp.minimum(offmap + (j + base), dump)
        valid = (d >= _A0) & (d <= _BMAX)
        out_ref[b] = jnp.where(valid, comb, dump)


def _make_idx_call(f, n, fb, nrep, dump):
    return pl.pallas_call(
        functools.partial(_idx_body, fb, nrep, dump),
        grid=(f // fb,),
        in_specs=[
            pl.BlockSpec(memory_space=pltpu.SMEM),
            pl.BlockSpec((fb, n, 3), lambda i: (i, 0, 0)),
            pl.BlockSpec((fb, 3, n), lambda i: (i, 0, 0)),
            pl.BlockSpec((n, n), lambda i: (0, 0)),
        ],
        out_specs=pl.BlockSpec((fb, n, n), lambda i: (i, 0, 0)),
        out_shape=jax.ShapeDtypeStruct((f, n, n), jnp.int32),
    )


def _sc_hist_body(perw, ch, accw, idx_hbm, out_hbm, acc_ref, buf_ref,
                  sem0, sem1):
    wid = lax.axis_index("s") * _NCORE + lax.axis_index("c")
    lane_off = lax.iota(jnp.int32, 16) * accw
    ones = jnp.ones((16,), jnp.float32)
    zeros = jnp.zeros((16,), jnp.float32)

    @plsc.parallel_loop(0, _NLANE * accw, 16, unroll=8)
    def _(i):
        acc_ref[pl.ds(i, 16)] = zeros

    nchunk = perw // ch
    base = wid * perw

    def copy_in(ci, slot, sem):
        return pltpu.async_copy(
            idx_hbm.at[pl.ds(base + ci * ch, ch)],
            buf_ref.at[pl.ds(slot * ch, ch)], sem)

    copy_in(0, 0, sem0)

    def chunk_body(oi, _):
        for b in range(2):
            ci = oi * 2 + b
            sem = sem0 if b == 0 else sem1
            nsem = sem1 if b == 0 else sem0

            @pl.when(ci + 1 < nchunk)
            def _():
                copy_in(ci + 1, 1 - b, nsem)

            pltpu.make_async_copy(
                idx_hbm.at[pl.ds(base, ch)],
                buf_ref.at[pl.ds(b * ch, ch)], sem).wait()

            @plsc.parallel_loop(0, ch, 16, unroll=10)
            def _(vi):
                v = buf_ref[pl.ds(b * ch + vi, 16)]
                plsc.addupdate_scatter(acc_ref, [lane_off + v], ones)
        return 0
    lax.fori_loop(0, nchunk // 2, chunk_body, 0)

    @plsc.parallel_loop(0, accw, 16, unroll=2)
    def _(i):
        s = acc_ref[pl.ds(i, 16)]
        for rr in range(1, _NLANE):
            s = s + acc_ref[pl.ds(rr * accw + i, 16)]
        acc_ref[pl.ds(i, 16)] = s

    pltpu.sync_copy(acc_ref.at[pl.ds(0, accw)],
                    out_hbm.at[pl.ds(wid * accw, accw)])


def _make_sc_hist(tot, accw):
    perw = tot // _NWORKER
    ch = 7200
    while perw % ch != 0 or (perw // ch) % 2 != 0:
        ch //= 2
    mesh = plsc.VectorSubcoreMesh(
        core_axis_name="c", subcore_axis_name="s", num_cores=_NCORE)
    return pl.kernel(
        functools.partial(_sc_hist_body, perw, ch, accw),
        out_type=jax.ShapeDtypeStruct((_NWORKER * accw,), jnp.float32),
        mesh=mesh,
        compiler_params=pltpu.CompilerParams(needs_layout_passes=False),
        scratch_types=[
            pltpu.VMEM((_NLANE * accw,), jnp.float32),
            pltpu.VMEM((2 * ch,), jnp.int32),
            pltpu.SemaphoreType.DMA,
            pltpu.SemaphoreType.DMA,
        ],
    )


def _norm_body(t, nrep, n_o, n_h, lat_ref, parts_ref, gts_ref, shell_ref,
               rdf_ref, maes_ref):
    tot = jnp.sum(parts_ref[...], axis=0, keepdims=True)
    prod_l = lat_ref[0] * lat_ref[1] * lat_ref[2]
    counts = (n_o * n_o, n_h * n_h, n_o * n_h)
    shell = shell_ref[...]
    li = lax.broadcasted_iota(jnp.int32, (1, 128), 1)
    mvec = jnp.zeros((1, 128), jnp.float32)
    for rep in range(nrep):
        maes = []
        for c in range(3):
            h = lax.slice(tot, (0, rep * 3 * _NB + c * _NB),
                          (1, rep * 3 * _NB + (c + 1) * _NB))
            data_shape = jnp.float32(t) * jnp.float32(counts[c])
            rho = data_shape / prod_l
            z = rho * shell
            rdf = h / z
            rdf_ref[pl.ds(rep * 3 + c, 1), :] = rdf
            g = gts_ref[pl.ds(c, 1), :]
            maes.append(_XLIM * (jnp.sum(jnp.abs(rdf - g)) / jnp.float32(_NB)))
        m = jnp.maximum(jnp.maximum(maes[0], maes[1]), maes[2])
        mvec = jnp.where(li == rep, m, mvec)
    maes_ref[...] = mvec


def _make_norm_call(t, nrep, n_o, n_h, accw):
    return pl.pallas_call(
        functools.partial(_norm_body, t, nrep, n_o, n_h),
        in_specs=[
            pl.BlockSpec(memory_space=pltpu.SMEM),
            pl.BlockSpec((_NWORKER, accw), lambda: (0, 0)),
            pl.BlockSpec((3, _NB), lambda: (0, 0)),
            pl.BlockSpec((1, _NB), lambda: (0, 0)),
        ],
        out_specs=[
            pl.BlockSpec((3 * nrep, _NB), lambda: (0, 0)),
            pl.BlockSpec((1, 128), lambda: (0, 0)),
        ],
        out_shape=[
            jax.ShapeDtypeStruct((3 * nrep, _NB), jnp.float32),
            jax.ShapeDtypeStruct((1, 128), jnp.float32),
        ],
    )


def kernel(stacked_radii, ptypes, lattices, gt_OO, gt_HH, gt_HO):
    t, nrep, n, _ = stacked_radii.shape
    f = t * nrep
    n_o = n // 3
    n_h = n - n_o
    dump = nrep * 3 * _NB
    accw = dump + (16 - dump % 16) % 16 + 16

    fb = 2 * nrep if f % (2 * nrep) == 0 else nrep

    pos = stacked_radii.reshape(f, n, 3)
    pos2 = jnp.concatenate([pos[:, 0::3], pos[:, 1::3], pos[:, 2::3]], axis=1)
    pcol = jnp.transpose(pos2, (0, 2, 1))

    ri = lax.broadcasted_iota(jnp.int32, (n, n), 0)
    ci = lax.broadcasted_iota(jnp.int32, (n, n), 1)
    is_oa = ri < n_o
    is_ob = ci < n_o
    offmap = jnp.where(
        is_oa & is_ob, 0,
        jnp.where((~is_oa) & (~is_ob), _NB,
                  jnp.where(is_oa & (~is_ob), 2 * _NB, 1 << 20)))

    idx = _make_idx_call(f, n, fb, nrep, dump)(lattices, pos2, pcol, offmap)
    parts = _make_sc_hist(f * n * n, accw)(idx.reshape(-1))
    parts = parts.reshape(_NWORKER, accw)

    gts = jnp.concatenate([gt_OO, gt_HH, gt_HO], axis=0)
    shell = jnp.asarray(_SHELL.astype(np.float32))[None, :]
    rdf12, maes_pad = _make_norm_call(t, nrep, n_o, n_h, accw)(
        lattices, parts, gts, shell)
    return rdf12.reshape(nrep, 3 * _NB), maes_pad[0, :nrep]


# speedup vs baseline: 3164.8878x; 1.1426x over previous
import functools

import numpy as np
import jax
import jax.numpy as jnp
from jax import lax
from jax.experimental import pallas as pl
from jax.experimental.pallas import tpu as pltpu
from jax.experimental.pallas import tpu_sc as plsc

_XLIM = 6.0
_NB = 600
_BINS = np.linspace(1e-06, _XLIM, _NB + 1).astype(np.float32)
_SHELL = (4.0 / 3.0) * np.pi * (_BINS[1:] ** 3 - _BINS[:-1] ** 3)
_H64 = (_XLIM - 1e-06) / _NB
_HS = np.float32(_H64)
_INV_H = np.float32(1.0 / _H64)
_A0 = np.float32(_BINS[0])
_BMAX = np.float32(_BINS[-1])

_NLANE = 16
_NCORE = 2
_NSUB = 16
_NWORKER = _NCORE * _NSUB


def _wrap_body(lat_ref, pcol_ref, out_ref):
    lrow = jnp.concatenate(
        [jnp.full((1, 1), lat_ref[k], jnp.float32) for k in range(3)],
        axis=0)[None]
    p = pcol_ref[...]
    out_ref[...] = (p / lrow % 1.0) * lrow


def _make_wrap_call(f, n):
    return pl.pallas_call(
        _wrap_body,
        in_specs=[
            pl.BlockSpec(memory_space=pltpu.SMEM),
            pl.BlockSpec((f, 3, n), lambda: (0, 0, 0)),
        ],
        out_specs=pl.BlockSpec((f, 3, n), lambda: (0, 0, 0)),
        out_shape=jax.ShapeDtypeStruct((f, 3, n), jnp.float32),
    )


def _idx_body(fb, nrep, dump, lat_ref, prow_ref, pcol_ref, off_ref, out_ref):
    n = prow_ref.shape[1]
    offmap = off_ref[...]
    for b in range(fb):
        acc = jnp.zeros((n, n), jnp.float32)
        for k in range(3):
            lk = lat_ref[k]
            pa = prow_ref[b, :, pl.ds(k, 1)]
            pb = pcol_ref[b, pl.ds(k, 1), :]
            t = jnp.abs(pa - pb)
            m = jnp.minimum(t, lk - t)
            acc = acc + m * m
        d = jnp.sqrt(acc)
        j = jnp.floor((d - _A0) * _INV_H).astype(jnp.int32)
        j = jnp.clip(j, 0, _NB - 1)
        base = (b % nrep) * (3 * _NB)
        comb = jnp.minimum(offmap + (j + base), dump)
        valid = (d >= _A0) & (d <= _BMAX)
        out_ref[b] = jnp.where(valid, comb, dump)


def _make_idx_call(f, n, fb, nrep, dump):
    return pl.pallas_call(
        functools.partial(_idx_body, fb, nrep, dump),
        grid=(f // fb,),
        in_specs=[
            pl.BlockSpec(memory_space=pltpu.SMEM),
            pl.BlockSpec((fb, n, 3), lambda i: (i, 0, 0)),
            pl.BlockSpec((fb, 3, n), lambda i: (i, 0, 0)),
            pl.BlockSpec((n, n), lambda i: (0, 0)),
        ],
        out_specs=pl.BlockSpec((fb, n, n), lambda i: (i, 0, 0)),
        out_shape=jax.ShapeDtypeStruct((f, n, n), jnp.int32),
    )


def _sc_hist_body(perw, ch, accw, idx_hbm, out_hbm, acc_ref, buf_ref,
                  sem0, sem1):
    wid = lax.axis_index("s") * _NCORE + lax.axis_index("c")
    lane_off = lax.iota(jnp.int32, 16) * accw
    ones = jnp.ones((16,), jnp.float32)
    zeros = jnp.zeros((16,), jnp.float32)

    @plsc.parallel_loop(0, _NLANE * accw, 16, unroll=8)
    def _(i):
        acc_ref[pl.ds(i, 16)] = zeros

    nchunk = perw // ch
    base = wid * perw

    def copy_in(ci, slot, sem):
        return pltpu.async_copy(
            idx_hbm.at[pl.ds(base + ci * ch, ch)],
            buf_ref.at[pl.ds(slot * ch, ch)], sem)

    copy_in(0, 0, sem0)

    def chunk_body(oi, _):
        for b in range(2):
            ci = oi * 2 + b
            sem = sem0 if b == 0 else sem1
            nsem = sem1 if b == 0 else sem0

            @pl.when(ci + 1 < nchunk)
            def _():
                copy_in(ci + 1, 1 - b, nsem)

            pltpu.make_async_copy(
                idx_hbm.at[pl.ds(base, ch)],
                buf_ref.at[pl.ds(b * ch, ch)], sem).wait()

            @plsc.parallel_loop(0, ch, 16, unroll=10)
            def _(vi):
                v = buf_ref[pl.ds(b * ch + vi, 16)]
                plsc.addupdate_scatter(acc_ref, [lane_off + v], ones)
        return 0
    lax.fori_loop(0, nchunk // 2, chunk_body, 0)

    @plsc.parallel_loop(0, accw, 16, unroll=2)
    def _(i):
        s = acc_ref[pl.ds(i, 16)]
        for rr in range(1, _NLANE):
            s = s + acc_ref[pl.ds(rr * accw + i, 16)]
        acc_ref[pl.ds(i, 16)] = s

    pltpu.sync_copy(acc_ref.at[pl.ds(0, accw)],
                    out_hbm.at[pl.ds(wid * accw, accw)])


def _make_sc_hist(tot, accw):
    perw = tot // _NWORKER
    ch = 7200
    while perw % ch != 0 or (perw // ch) % 2 != 0:
        ch //= 2
    mesh = plsc.VectorSubcoreMesh(
        core_axis_name="c", subcore_axis_name="s", num_cores=_NCORE)
    return pl.kernel(
        functools.partial(_sc_hist_body, perw, ch, accw),
        out_type=jax.ShapeDtypeStruct((_NWORKER * accw,), jnp.float32),
        mesh=mesh,
        compiler_params=pltpu.CompilerParams(needs_layout_passes=False),
        scratch_types=[
            pltpu.VMEM((_NLANE * accw,), jnp.float32),
            pltpu.VMEM((2 * ch,), jnp.int32),
            pltpu.SemaphoreType.DMA,
            pltpu.SemaphoreType.DMA,
        ],
    )


def _norm_body(t, nrep, n_o, n_h, lat_ref, parts_ref, gts_ref, shell_ref,
               rdf_ref, maes_ref):
    tot = jnp.sum(parts_ref[...], axis=0, keepdims=True)
    prod_l = lat_ref[0] * lat_ref[1] * lat_ref[2]
    counts = (n_o * n_o, n_h * n_h, n_o * n_h)
    shell = shell_ref[...]
    li = lax.broadcasted_iota(jnp.int32, (1, 128), 1)
    mvec = jnp.zeros((1, 128), jnp.float32)
    for rep in range(nrep):
        maes = []
        for c in range(3):
            h = lax.slice(tot, (0, rep * 3 * _NB + c * _NB),
                          (1, rep * 3 * _NB + (c + 1) * _NB))
            data_shape = jnp.float32(t) * jnp.float32(counts[c])
            rho = data_shape / prod_l
            z = rho * shell
            rdf = h / z
            rdf_ref[pl.ds(rep * 3 + c, 1), :] = rdf
            g = gts_ref[pl.ds(c, 1), :]
            maes.append(_XLIM * (jnp.sum(jnp.abs(rdf - g)) / jnp.float32(_NB)))
        m = jnp.maximum(jnp.maximum(maes[0], maes[1]), maes[2])
        mvec = jnp.where(li == rep, m, mvec)
    maes_ref[...] = mvec


def _make_norm_call(t, nrep, n_o, n_h, accw):
    return pl.pallas_call(
        functools.partial(_norm_body, t, nrep, n_o, n_h),
        in_specs=[
            pl.BlockSpec(memory_space=pltpu.SMEM),
            pl.BlockSpec((_NWORKER, accw), lambda: (0, 0)),
            pl.BlockSpec((3, _NB), lambda: (0, 0)),
            pl.BlockSpec((1, _NB), lambda: (0, 0)),
        ],
        out_specs=[
            pl.BlockSpec((3 * nrep, _NB), lambda: (0, 0)),
            pl.BlockSpec((1, 128), lambda: (0, 0)),
        ],
        out_shape=[
            jax.ShapeDtypeStruct((3 * nrep, _NB), jnp.float32),
            jax.ShapeDtypeStruct((1, 128), jnp.float32),
        ],
    )


def kernel(stacked_radii, ptypes, lattices, gt_OO, gt_HH, gt_HO):
    t, nrep, n, _ = stacked_radii.shape
    f = t * nrep
    n_o = n // 3
    n_h = n - n_o
    dump = nrep * 3 * _NB
    accw = dump + (16 - dump % 16) % 16 + 16

    fb = 2 * nrep if f % (2 * nrep) == 0 else nrep

    pos = stacked_radii.reshape(f, n, 3)
    pos2 = jnp.concatenate([pos[:, 0::3], pos[:, 1::3], pos[:, 2::3]], axis=1)
    pcol = _make_wrap_call(f, n)(lattices, jnp.transpose(pos2, (0, 2, 1)))
    pos2 = jnp.transpose(pcol, (0, 2, 1))

    ri = lax.broadcasted_iota(jnp.int32, (n, n), 0)
    ci = lax.broadcasted_iota(jnp.int32, (n, n), 1)
    is_oa = ri < n_o
    is_ob = ci < n_o
    offmap = jnp.where(
        is_oa & is_ob, 0,
        jnp.where((~is_oa) & (~is_ob), _NB,
                  jnp.where(is_oa & (~is_ob), 2 * _NB, 1 << 20)))

    idx = _make_idx_call(f, n, fb, nrep, dump)(lattices, pos2, pcol, offmap)
    parts = _make_sc_hist(f * n * n, accw)(idx.reshape(-1))
    parts = parts.reshape(_NWORKER, accw)

    gts = jnp.concatenate([gt_OO, gt_HH, gt_HO], axis=0)
    shell = jnp.asarray(_SHELL.astype(np.float32))[None, :]
    rdf12, maes_pad = _make_norm_call(t, nrep, n_o, n_h, accw)(
        lattices, parts, gts, shell)
    return rdf12.reshape(nrep, 3 * _NB), maes_pad[0, :nrep]


# speedup vs baseline: 3823.6850x; 1.2082x over previous
import functools

import numpy as np
import jax
import jax.numpy as jnp
from jax import lax
from jax.experimental import pallas as pl
from jax.experimental.pallas import tpu as pltpu
from jax.experimental.pallas import tpu_sc as plsc

_XLIM = 6.0
_NB = 600
_BINS = np.linspace(1e-06, _XLIM, _NB + 1).astype(np.float32)
_SHELL = (4.0 / 3.0) * np.pi * (_BINS[1:] ** 3 - _BINS[:-1] ** 3)
_H64 = (_XLIM - 1e-06) / _NB
_HS = np.float32(_H64)
_INV_H = np.float32(1.0 / _H64)
_A0 = np.float32(_BINS[0])
_BMAX = np.float32(_BINS[-1])

_NLANE = 16
_NCORE = 2
_NSUB = 16
_NWORKER = _NCORE * _NSUB


def _wrap_body(lat_ref, pcol_ref, out_ref):
    lrow = jnp.concatenate(
        [jnp.full((1, 1), lat_ref[k], jnp.float32) for k in range(3)],
        axis=0)[None]
    p = pcol_ref[...]
    out_ref[...] = (p / lrow % 1.0) * lrow


def _make_wrap_call(f, n):
    return pl.pallas_call(
        _wrap_body,
        in_specs=[
            pl.BlockSpec(memory_space=pltpu.SMEM),
            pl.BlockSpec((f, 3, n), lambda: (0, 0, 0)),
        ],
        out_specs=pl.BlockSpec((f, 3, n), lambda: (0, 0, 0)),
        out_shape=jax.ShapeDtypeStruct((f, 3, n), jnp.float32),
    )


def _pair_idx(lat_ref, prow_ref, pcol_ref, b, rs, rn, cs, cn, base):
    acc = jnp.zeros((rn, cn), jnp.float32)
    for k in range(3):
        lk = lat_ref[k]
        pa = prow_ref[b, pl.ds(rs, rn), pl.ds(k, 1)]
        pb = pcol_ref[b, pl.ds(k, 1), pl.ds(cs, cn)]
        t = jnp.abs(pa - pb)
        m = jnp.minimum(t, lk - t)
        acc = acc + m * m
    d = jnp.sqrt(acc)
    j = jnp.floor((d - _A0) * _INV_H).astype(jnp.int32)
    j = jnp.clip(j, 0, _NB - 1) + base
    valid = (d >= _A0) & (d <= _BMAX)
    return j, valid


def _idx_body(fb, nrep, n_o, dump, lat_ref, prow_ref, pcol_ref, off_ref,
              out1_ref, out2_ref):
    n = prow_ref.shape[1]
    n_h = n - n_o
    offmap = off_ref[...]
    for b in range(fb):
        base = (b % nrep) * (3 * _NB)
        j1, v1 = _pair_idx(lat_ref, prow_ref, pcol_ref, b, 0, n_o, 0, n, base)
        out1_ref[b] = jnp.where(v1, offmap + j1, dump)
        j2, v2 = _pair_idx(lat_ref, prow_ref, pcol_ref, b, n_o, n_h, n_o, n_h,
                           base + _NB)
        out2_ref[b] = jnp.where(v2, j2, dump)


def _make_idx_call(f, n, fb, nrep, n_o, dump):
    n_h = n - n_o
    return pl.pallas_call(
        functools.partial(_idx_body, fb, nrep, n_o, dump),
        grid=(f // fb,),
        in_specs=[
            pl.BlockSpec(memory_space=pltpu.SMEM),
            pl.BlockSpec((fb, n, 3), lambda i: (i, 0, 0)),
            pl.BlockSpec((fb, 3, n), lambda i: (i, 0, 0)),
            pl.BlockSpec((n_o, n), lambda i: (0, 0)),
        ],
        out_specs=[
            pl.BlockSpec((fb, n_o, n), lambda i: (i, 0, 0)),
            pl.BlockSpec((fb, n_h, n_h), lambda i: (i, 0, 0)),
        ],
        out_shape=[
            jax.ShapeDtypeStruct((f, n_o, n), jnp.int32),
            jax.ShapeDtypeStruct((f, n_h, n_h), jnp.int32),
        ],
    )


def _sc_hist_body(perw1, perw2, ch, accw, idx1_hbm, idx2_hbm, out_hbm,
                  acc_ref, buf_ref, sem0, sem1):
    wid = lax.axis_index("s") * _NCORE + lax.axis_index("c")
    lane_off = lax.iota(jnp.int32, 16) * accw
    ones = jnp.ones((16,), jnp.float32)
    zeros = jnp.zeros((16,), jnp.float32)

    @plsc.parallel_loop(0, _NLANE * accw, 16, unroll=8)
    def _(i):
        acc_ref[pl.ds(i, 16)] = zeros

    def run_stream(idx_hbm, perw):
        nchunk = perw // ch
        base = wid * perw

        def copy_in(ci, slot, sem):
            return pltpu.async_copy(
                idx_hbm.at[pl.ds(base + ci * ch, ch)],
                buf_ref.at[pl.ds(slot * ch, ch)], sem)

        copy_in(0, 0, sem0)

        def chunk_body(oi, _):
            for b in range(2):
                ci = oi * 2 + b
                sem = sem0 if b == 0 else sem1
                nsem = sem1 if b == 0 else sem0

                @pl.when(ci + 1 < nchunk)
                def _():
                    copy_in(ci + 1, 1 - b, nsem)

                pltpu.make_async_copy(
                    idx_hbm.at[pl.ds(base, ch)],
                    buf_ref.at[pl.ds(b * ch, ch)], sem).wait()

                @plsc.parallel_loop(0, ch, 16, unroll=10)
                def _(vi):
                    v = buf_ref[pl.ds(b * ch + vi, 16)]
                    plsc.addupdate_scatter(acc_ref, [lane_off + v], ones)
            return 0
        lax.fori_loop(0, nchunk // 2, chunk_body, 0)

    run_stream(idx1_hbm, perw1)
    run_stream(idx2_hbm, perw2)

    @plsc.parallel_loop(0, accw, 16, unroll=2)
    def _(i):
        s = acc_ref[pl.ds(i, 16)]
        for rr in range(1, _NLANE):
            s = s + acc_ref[pl.ds(rr * accw + i, 16)]
        acc_ref[pl.ds(i, 16)] = s

    pltpu.sync_copy(acc_ref.at[pl.ds(0, accw)],
                    out_hbm.at[pl.ds(wid * accw, accw)])


def _make_sc_hist(tot1, tot2, accw):
    perw1 = tot1 // _NWORKER
    perw2 = tot2 // _NWORKER
    ch = 6400
    while (perw1 % ch != 0 or (perw1 // ch) % 2 != 0
           or perw2 % ch != 0 or (perw2 // ch) % 2 != 0):
        ch //= 2
    mesh = plsc.VectorSubcoreMesh(
        core_axis_name="c", subcore_axis_name="s", num_cores=_NCORE)
    return pl.kernel(
        functools.partial(_sc_hist_body, perw1, perw2, ch, accw),
        out_type=jax.ShapeDtypeStruct((_NWORKER * accw,), jnp.float32),
        mesh=mesh,
        compiler_params=pltpu.CompilerParams(needs_layout_passes=False),
        scratch_types=[
            pltpu.VMEM((_NLANE * accw,), jnp.float32),
            pltpu.VMEM((2 * ch,), jnp.int32),
            pltpu.SemaphoreType.DMA,
            pltpu.SemaphoreType.DMA,
        ],
    )


def _norm_body(t, nrep, n_o, n_h, lat_ref, parts_ref, gts_ref, shell_ref,
               rdf_ref, maes_ref):
    tot = jnp.sum(parts_ref[...], axis=0, keepdims=True)
    prod_l = lat_ref[0] * lat_ref[1] * lat_ref[2]
    counts = (n_o * n_o, n_h * n_h, n_o * n_h)
    shell = shell_ref[...]
    li = lax.broadcasted_iota(jnp.int32, (1, 128), 1)
    mvec = jnp.zeros((1, 128), jnp.float32)
    for rep in range(nrep):
        maes = []
        for c in range(3):
            h = lax.slice(tot, (0, rep * 3 * _NB + c * _NB),
                          (1, rep * 3 * _NB + (c + 1) * _NB))
            data_shape = jnp.float32(t) * jnp.float32(counts[c])
            rho = data_shape / prod_l
            z = rho * shell
            rdf = h / z
            rdf_ref[pl.ds(rep * 3 + c, 1), :] = rdf
            g = gts_ref[pl.ds(c, 1), :]
            maes.append(_XLIM * (jnp.sum(jnp.abs(rdf - g)) / jnp.float32(_NB)))
        m = jnp.maximum(jnp.maximum(maes[0], maes[1]), maes[2])
        mvec = jnp.where(li == rep, m, mvec)
    maes_ref[...] = mvec


def _make_norm_call(t, nrep, n_o, n_h, accw):
    return pl.pallas_call(
        functools.partial(_norm_body, t, nrep, n_o, n_h),
        in_specs=[
            pl.BlockSpec(memory_space=pltpu.SMEM),
            pl.BlockSpec((_NWORKER, accw), lambda: (0, 0)),
            pl.BlockSpec((3, _NB), lambda: (0, 0)),
            pl.BlockSpec((1, _NB), lambda: (0, 0)),
        ],
        out_specs=[
            pl.BlockSpec((3 * nrep, _NB), lambda: (0, 0)),
            pl.BlockSpec((1, 128), lambda: (0, 0)),
        ],
        out_shape=[
            jax.ShapeDtypeStruct((3 * nrep, _NB), jnp.float32),
            jax.ShapeDtypeStruct((1, 128), jnp.float32),
        ],
    )


def kernel(stacked_radii, ptypes, lattices, gt_OO, gt_HH, gt_HO):
    t, nrep, n, _ = stacked_radii.shape
    f = t * nrep
    n_o = n // 3
    n_h = n - n_o
    dump = nrep * 3 * _NB
    accw = dump + (16 - dump % 16) % 16 + 16

    fb = 2 * nrep if f % (2 * nrep) == 0 else nrep

    pos = stacked_radii.reshape(f, n, 3)
    pos2 = jnp.concatenate([pos[:, 0::3], pos[:, 1::3], pos[:, 2::3]], axis=1)
    pcol = _make_wrap_call(f, n)(lattices, jnp.transpose(pos2, (0, 2, 1)))
    pos2 = jnp.transpose(pcol, (0, 2, 1))

    ci = lax.broadcasted_iota(jnp.int32, (n_o, n), 1)
    offmap = jnp.where(ci < n_o, 0, 2 * _NB)

    idx1, idx2 = _make_idx_call(f, n, fb, nrep, n_o, dump)(
        lattices, pos2, pcol, offmap)
    parts = _make_sc_hist(f * n_o * n, f * n_h * n_h, accw)(
        idx1.reshape(-1), idx2.reshape(-1))
    parts = parts.reshape(_NWORKER, accw)

    gts = jnp.concatenate([gt_OO, gt_HH, gt_HO], axis=0)
    shell = jnp.asarray(_SHELL.astype(np.float32))[None, :]
    rdf12, maes_pad = _make_norm_call(t, nrep, n_o, n_h, accw)(
        lattices, parts, gts, shell)
    return rdf12.reshape(nrep, 3 * _NB), maes_pad[0, :nrep]
